# baseline (speedup 1.0000x reference)
import functools

import jax
import jax.numpy as jnp
from jax import lax
from jax.experimental import pallas as pl
from jax.experimental.pallas import tpu as pltpu
from jax.experimental.pallas import tpu_sc as plsc

N_ROWS = 100000
D = 128
NUM_CORES = 2
NUM_SUBCORES = 16
NW = NUM_CORES * NUM_SUBCORES
CHUNK = 200
NCHUNKS = N_ROWS // CHUNK
NMAX = (NCHUNKS + NW - 1) // NW
NBUF = 4

_mesh = plsc.VectorSubcoreMesh(core_axis_name="c", subcore_axis_name="s")


@functools.partial(
    pl.kernel,
    mesh=_mesh,
    out_type=jax.ShapeDtypeStruct((N_ROWS, D), jnp.float32),
    scratch_types=(
        [pltpu.VMEM((CHUNK,), jnp.int32) for _ in range(NBUF)]
        + [pltpu.VMEM((CHUNK, D), jnp.float32) for _ in range(NBUF)]
        + [pltpu.SemaphoreType.DMA for _ in range(2 * NBUF)]
    ),
)
def _gather(table_hbm, idx_hbm, out_hbm, *scratch):
    idxs = scratch[:NBUF]
    rows = scratch[NBUF:2 * NBUF]
    gsems = scratch[2 * NBUF:3 * NBUF]
    wsems = scratch[3 * NBUF:]
    wid = lax.axis_index("s") * NUM_CORES + lax.axis_index("c")

    def chunk_id(t):
        return wid + t * NW

    def start_gather(t):
        b = t % NBUF
        c = chunk_id(t)

        @pl.when(c < NCHUNKS)
        def _():
            pltpu.sync_copy(idx_hbm.at[pl.ds(c * CHUNK, CHUNK)], idxs[b])
            pltpu.async_copy(table_hbm.at[idxs[b]], rows[b], gsems[b])

    def start_write(t):
        b = t % NBUF
        c = chunk_id(t)

        @pl.when(c < NCHUNKS)
        def _():
            pltpu.make_async_copy(table_hbm.at[idxs[b]], rows[b],
                                  gsems[b]).wait()
            pltpu.async_copy(rows[b], out_hbm.at[pl.ds(c * CHUNK, CHUNK)],
                             wsems[b])

    def wait_write(t):
        b = t % NBUF
        c = chunk_id(t)

        @pl.when(c < NCHUNKS)
        def _():
            pltpu.make_async_copy(
                rows[b], out_hbm.at[pl.ds(c * CHUNK, CHUNK)], wsems[b]).wait()

    for s in range(NBUF - 1):
        start_gather(s)
    for t in range(NMAX):
        s = t + NBUF - 1
        if s < NMAX:
            if s - NBUF >= 0:
                wait_write(s - NBUF)
            start_gather(s)
        start_write(t)
    for t in range(max(0, NMAX - NBUF), NMAX):
        wait_write(t)


def kernel(g, h, r, norm, W):
    idx = h.reshape(-1).astype(jnp.int32)
    return _gather(W, idx)
